# baseline (speedup 1.0000x reference)
import jax
import jax.numpy as jnp
from jax import lax
from jax.experimental import pallas as pl
from jax.experimental.pallas import tpu as pltpu
from jax.experimental.pallas import tpu_sc as plsc

NC, NS, L = 2, 16, 16
NW = NC * NS
W = 128

CB, CA, CT, CI, CN = 320, 640, 944, 160, 1024


def _rsqrt(x):
    xi = lax.bitcast_convert_type(x, jnp.int32)
    y = lax.bitcast_convert_type(jnp.int32(0x5F3759DF) - (xi >> 1),
                                 jnp.float32)
    for _ in range(3):
        y = y * (1.5 - 0.5 * x * y * y)
    return y


def _cross(ax, ay, az, bx, by, bz):
    return (ay * bz - az * by, az * bx - ax * bz, ax * by - ay * bx)


def _e_bond(g, p):
    (x0, y0, z0), (x1, y1, z1) = g
    r0, k = p
    dx = x0 - x1; dy = y0 - y1; dz = z0 - z1
    d2 = dx * dx + dy * dy + dz * dz
    dist = d2 * _rsqrt(jnp.maximum(d2, 1e-30))
    s = dist - r0
    return 0.5 * k * s * s


def _e_angle(g, p):
    (x0, y0, z0), (x1, y1, z1), (x2, y2, z2) = g
    kf, c0, c1, c2, order = p
    v1x = x0 - x1; v1y = y0 - y1; v1z = z0 - z1
    v2x = x2 - x1; v2y = y2 - y1; v2z = z2 - z1
    d1 = v1x * v1x + v1y * v1y + v1z * v1z
    d2 = v2x * v2x + v2y * v2y + v2z * v2z
    dot = v1x * v2x + v1y * v2y + v1z * v2z
    ct = dot * _rsqrt(jnp.maximum(d1 * d2, 1e-24))
    ct = jnp.clip(ct, -0.999999, 0.999999)
    ss = jnp.maximum(1.0 - ct * ct, 1e-12)
    c2t = ct * ct - ss
    base = c0 + c1 * ct + c2 * c2t
    t3 = ct * (ct * ct - 3.0 * ss)
    t4 = ct * ct * ct * ct - 6.0 * ct * ct * ss + ss * ss
    terms = jnp.where(order == 1, -ct,
            jnp.where(order == 2, c2t,
            jnp.where(order == 3, t3,
            jnp.where(order == 4, t4, 0.0))))
    of = order.astype(jnp.float32)
    alt = (1.0 - terms) / jnp.maximum(of * of, 1.0)
    return kf * jnp.where(order > 0, alt, base)


def _e_torsion(g, p):
    (x1, y1, z1), (x2, y2, z2), (x3, y3, z3), (x4, y4, z4) = g
    kf, order, cos_t = p
    r1x = x1 - x2; r1y = y1 - y2; r1z = z1 - z2
    r2x = x3 - x2; r2y = y3 - y2; r2z = z3 - z2
    r4x = x4 - x3; r4y = y4 - y3; r4z = z4 - z3
    tax, tay, taz = _cross(r1x, r1y, r1z, r2x, r2y, r2z)
    tbx, tby, tbz = _cross(-r2x, -r2y, -r2z, r4x, r4y, r4z)
    da2 = tax * tax + tay * tay + taz * taz
    db2 = tbx * tbx + tby * tby + tbz * tbz
    dot = tax * tbx + tay * tby + taz * tbz
    cp = dot * _rsqrt(jnp.maximum(da2 * db2, 1e-24))
    cp = jnp.clip(cp, -0.999999, 0.999999)
    sst = jnp.maximum(1.0 - cp * cp, 1e-12)
    cn2 = 1.0 - 2.0 * sst
    cn3 = cp * (cp * cp - 3.0 * sst)
    cn6 = 1.0 + sst * ((-32.0 * sst + 48.0) * sst - 18.0)
    cn = jnp.where(order == 2, cn2,
         jnp.where(order == 3, cn3,
         jnp.where(order == 6, cn6, 0.0)))
    return 0.5 * kf * (1.0 - cos_t * cn)


def _e_inversion(g, p):
    (xc, yc, zc), (xa, ya, za), (xb, yb, zb), (xd, yd, zd) = g
    kf, c0, c1, c2 = p
    rax = xa - xc; ray = ya - yc; raz = za - zc
    rbx = xb - xc; rby = yb - yc; rbz = zb - zc
    rdx = xd - xc; rdy = yd - yc; rdz = zd - zc
    nx, ny, nz = _cross(rax, ray, raz, rbx, rby, rbz)
    nn = nx * nx + ny * ny + nz * nz
    dd = rdx * rdx + rdy * rdy + rdz * rdz
    dot = nx * rdx + ny * rdy + nz * rdz
    sw = dot * _rsqrt(jnp.maximum(nn * dd, 1e-24))
    sw = jnp.clip(sw, -0.999999, 0.999999)
    z = jnp.maximum(1.0 - sw * sw, 1e-12)
    cw = z * _rsqrt(z)
    c2w = 2.0 * cw * cw - 1.0
    return kf * (c0 + c1 * cw + c2 * c2w)


def _e_vdw(g, p):
    (x0, y0, z0), (x1, y1, z1) = g
    rm, eps, thr = p
    dx = x0 - x1; dy = y0 - y1; dz = z0 - z1
    d2 = dx * dx + dy * dy + dz * dz
    r2c = jnp.maximum(d2, 0.01)
    t = rm * _rsqrt(r2c)
    x2 = t * t
    x6 = x2 * x2 * x2
    ev = eps * (x6 * x6 - 2.0 * x6)
    return jnp.where(r2c <= thr * thr, ev, 0.0)


def _pad_to(n, cw):
    q = 2 * NW * cw
    return ((n + q - 1) // q) * q


def _prep_idx(col, tot, n_atoms):
    pad = tot - col.shape[0]
    filler = jnp.arange(pad, dtype=jnp.int32) % n_atoms
    return jnp.concatenate([col.astype(jnp.int32), filler])


def _prep_par(a, tot):
    pad = tot - a.shape[0]
    return jnp.concatenate([a, jnp.zeros((pad,), a.dtype)])


_SECTIONS = (
    (2, 2, 0, CB, _e_bond),
    (3, 4, 1, CA, _e_angle),
    (4, 2, 1, CT, lambda g, p: _e_torsion(g, (p[0], p[2], p[1]))),
    (4, 4, 0, CI, _e_inversion),
    (2, 3, 0, CN, _e_vdw),
)


def _mk_kernel(counts, npad):
    nks = [_pad_to(n, s[3]) // (NW * s[3]) for n, s in zip(counts, _SECTIONS)]
    mesh = plsc.VectorSubcoreMesh(
        core_axis_name="c", subcore_axis_name="s",
        num_cores=NC, num_subcores=NS)

    def body(*refs):
        xs, ys, zs = refs[0:3]
        pos = 3
        sec_in = []
        for ni, nf, nint, _, _ in _SECTIONS:
            k = 2 + nint
            sec_in.append(refs[pos:pos + k])
            pos += k
        out = refs[pos]; pos += 1
        sec_scr = []
        for ni, nf, nint, _, _ in _SECTIONS:
            k = 2 * (5 + nint)
            sec_scr.append(refs[pos:pos + k])
            pos += k
        xsh, ysh, zsh = refs[pos:pos + 3]; pos += 3
        bounce = refs[pos]; pos += 1
        acc_v = refs[pos]; pos += 1
        sems = refs[pos:pos + 2]

        sid = lax.axis_index("s")
        wid = sid * NC + lax.axis_index("c")
        acc = jnp.zeros((L,), jnp.float32)

        npad = xsh.shape[0]
        stripe = npad // NS
        s0 = sid * stripe
        for h, v in ((xs, xsh), (ys, ysh), (zs, zsh)):
            pltpu.sync_copy(h.at[pl.ds(s0, stripe)], bounce)
            pltpu.sync_copy(bounce, v.at[pl.ds(s0, stripe)])
        plsc.subcore_barrier()

        for (ni, nf, nint, cw, efn), ins, scr, nk in zip(
                _SECTIONS, sec_in, sec_scr, nks):
            idx_h = ins[0]
            parf_h = ins[1]
            pari_h = ins[2:]
            half = 5 + nint
            sets = []
            for p in (0, 1):
                s = scr[p * half:(p + 1) * half]
                sets.append((s[0], s[1:4], s[4], s[5:]))
            tile_ck = wid * nk

            def fire_l(ci, p, idx_h=idx_h, parf_h=parf_h, pari_h=pari_h,
                       sets=sets, cw=cw, ni=ni, nf=nf, tile_ck=tile_ck):
                gc = tile_ck + ci
                idx_v, _, parf_v, pari_v = sets[p]
                pltpu.async_copy(
                    idx_h.at[pl.ds(gc * (ni * cw), ni * cw)], idx_v, sems[p])
                pltpu.async_copy(
                    parf_h.at[pl.ds(gc * (nf * cw), nf * cw)], parf_v,
                    sems[p])
                for h, v in zip(pari_h, pari_v):
                    pltpu.async_copy(h.at[pl.ds(gc * cw, cw)], v, sems[p])

            def wait_l(p, idx_h=idx_h, parf_h=parf_h, pari_h=pari_h,
                       sets=sets, cw=cw, ni=ni, nf=nf):
                idx_v, _, parf_v, pari_v = sets[p]
                pltpu.make_async_copy(
                    idx_h.at[pl.ds(0, ni * cw)], idx_v, sems[p]).wait()
                pltpu.make_async_copy(
                    parf_h.at[pl.ds(0, nf * cw)], parf_v, sems[p]).wait()
                for h, v in zip(pari_h, pari_v):
                    pltpu.make_async_copy(
                        h.at[pl.ds(0, cw)], v, sems[p]).wait()

            def compute(p, acc_c, sets=sets, cw=cw, ni=ni, nf=nf, efn=efn):
                _, g_v, parf_v, pari_v = sets[p]

                def slc(s, acc_r):
                    off = s * L
                    sites = tuple(
                        (g_v[0][pl.ds(j * cw + off, L)],
                         g_v[1][pl.ds(j * cw + off, L)],
                         g_v[2][pl.ds(j * cw + off, L)])
                        for j in range(ni))
                    pars = tuple(parf_v[pl.ds(q * cw + off, L)]
                                 for q in range(nf))
                    pars += tuple(v[pl.ds(off, L)] for v in pari_v)
                    return acc_r + efn(sites, pars)

                return lax.fori_loop(0, cw // L, slc, acc_c)

            fire_l(0, 0)
            fire_l(1, 1)
            nt = nk // 2

            def pair(t, acc_p, sets=sets, nk=nk):
                gds = []
                for p in (0, 1):
                    wait_l(p)
                    idx_v, g_v, _, _ = sets[p]
                    g = [pltpu.async_copy(tbl.at[idx_v], g_v[c], sems[p])
                         for c, tbl in enumerate((xsh, ysh, zsh))]
                    gds.append(g)
                for p in (0, 1):
                    for d in gds[p]:
                        d.wait()
                    acc_p = compute(p, acc_p)
                    ci_next = jnp.minimum(2 * t + p + 2, nk - 1)
                    fire_l(ci_next, p)
                return acc_p

            acc = lax.fori_loop(0, nt, pair, acc)
            wait_l(0)
            wait_l(1)

        acc_v[...] = acc
        pltpu.sync_copy(acc_v, out.at[wid])

    scratch = []
    for ni, nf, nint, cw, _ in _SECTIONS:
        for _ in range(2):
            scratch += [pltpu.VMEM((ni * cw,), jnp.int32)]
            scratch += [pltpu.VMEM((ni * cw,), jnp.float32) for _ in range(3)]
            scratch += [pltpu.VMEM((nf * cw,), jnp.float32)]
            scratch += [pltpu.VMEM((cw,), jnp.int32) for _ in range(nint)]
    scratch += [pltpu.VMEM_SHARED((npad,), jnp.float32) for _ in range(3)]
    scratch += [pltpu.VMEM((npad // NS,), jnp.float32)]
    scratch += [pltpu.VMEM((L,), jnp.float32)]
    scratch += [pltpu.SemaphoreType.DMA, pltpu.SemaphoreType.DMA]

    return pl.kernel(
        body,
        out_type=jax.ShapeDtypeStruct((NW, L), jnp.float32),
        mesh=mesh,
        scratch_types=scratch,
    )


def kernel(coords, bond_index, bond_rest_length, bond_force_constant,
           angle_index, angle_force_constant, angle_c0, angle_c1, angle_c2,
           angle_order, torsion_index, torsion_force_constant, torsion_order,
           torsion_cos_term, inversion_index, inversion_force_constant,
           inversion_c0, inversion_c1, inversion_c2, nonbond_index,
           vdw_minimum, vdw_well_depth, vdw_threshold):
    n_atoms = coords.shape[0]
    npad = ((n_atoms + 8 * NS - 1) // (8 * NS)) * (8 * NS)

    sec_arrays = (
        (bond_index, (bond_rest_length, bond_force_constant)),
        (angle_index, (angle_force_constant, angle_c0, angle_c1, angle_c2,
                       angle_order)),
        (torsion_index, (torsion_force_constant, torsion_cos_term,
                         torsion_order)),
        (inversion_index, (inversion_force_constant, inversion_c0,
                           inversion_c1, inversion_c2)),
        (nonbond_index, (vdw_minimum, vdw_well_depth, vdw_threshold)),
    )

    args = [_prep_par(coords[:, 0], npad),
            _prep_par(coords[:, 1], npad),
            _prep_par(coords[:, 2], npad)]
    counts = []
    for (idx2d, params), (ni, nf, nint, cw, _) in zip(sec_arrays, _SECTIONS):
        n = idx2d.shape[0]
        counts.append(n)
        tot = _pad_to(n, cw)
        nck = tot // cw
        cols = jnp.stack([_prep_idx(idx2d[:, j], tot, n_atoms)
                          for j in range(ni)])
        args.append(cols.reshape(ni, nck, cw).transpose(1, 0, 2).reshape(-1))
        fpars = params[:len(params) - nint]
        ipars = params[len(params) - nint:]
        fp = jnp.stack([_prep_par(par, tot) for par in fpars])
        nf_ = len(fpars)
        args.append(fp.reshape(nf_, nck, cw).transpose(1, 0, 2).reshape(-1))
        for par in ipars:
            args.append(_prep_par(par, tot))

    out = _mk_kernel(tuple(counts), npad)(*args)
    return jnp.sum(out)


# speedup vs baseline: 1.4656x; 1.4656x over previous
import jax
import jax.numpy as jnp
from jax import lax
from jax.experimental import pallas as pl
from jax.experimental.pallas import tpu as pltpu
from jax.experimental.pallas import tpu_sc as plsc

NC, NS, L = 2, 16, 16
NW = NC * NS
W = 128

CB, CA, CT, CI, CN = 320, 640, 944, 160, 1024


def _rsqrt(x):
    xi = lax.bitcast_convert_type(x, jnp.int32)
    y = lax.bitcast_convert_type(jnp.int32(0x5F3759DF) - (xi >> 1),
                                 jnp.float32)
    for _ in range(3):
        y = y * (1.5 - 0.5 * x * y * y)
    return y


def _cross(ax, ay, az, bx, by, bz):
    return (ay * bz - az * by, az * bx - ax * bz, ax * by - ay * bx)


def _e_bond(g, p):
    (x0, y0, z0), (x1, y1, z1) = g
    r0, k = p
    dx = x0 - x1; dy = y0 - y1; dz = z0 - z1
    d2 = dx * dx + dy * dy + dz * dz
    dist = d2 * _rsqrt(jnp.maximum(d2, 1e-30))
    s = dist - r0
    return 0.5 * k * s * s


def _e_angle(g, p):
    (x0, y0, z0), (x1, y1, z1), (x2, y2, z2) = g
    kf, c0, c1, c2, order = p
    v1x = x0 - x1; v1y = y0 - y1; v1z = z0 - z1
    v2x = x2 - x1; v2y = y2 - y1; v2z = z2 - z1
    d1 = v1x * v1x + v1y * v1y + v1z * v1z
    d2 = v2x * v2x + v2y * v2y + v2z * v2z
    dot = v1x * v2x + v1y * v2y + v1z * v2z
    ct = dot * _rsqrt(jnp.maximum(d1 * d2, 1e-24))
    ct = jnp.clip(ct, -0.999999, 0.999999)
    ss = jnp.maximum(1.0 - ct * ct, 1e-12)
    c2t = ct * ct - ss
    base = c0 + c1 * ct + c2 * c2t
    t3 = ct * (ct * ct - 3.0 * ss)
    t4 = ct * ct * ct * ct - 6.0 * ct * ct * ss + ss * ss
    terms = jnp.where(order == 1, -ct,
            jnp.where(order == 2, c2t,
            jnp.where(order == 3, t3,
            jnp.where(order == 4, t4, 0.0))))
    of = order.astype(jnp.float32)
    alt = (1.0 - terms) / jnp.maximum(of * of, 1.0)
    return kf * jnp.where(order > 0, alt, base)


def _e_torsion(g, p):
    (x1, y1, z1), (x2, y2, z2), (x3, y3, z3), (x4, y4, z4) = g
    kf, order, cos_t = p
    r1x = x1 - x2; r1y = y1 - y2; r1z = z1 - z2
    r2x = x3 - x2; r2y = y3 - y2; r2z = z3 - z2
    r4x = x4 - x3; r4y = y4 - y3; r4z = z4 - z3
    tax, tay, taz = _cross(r1x, r1y, r1z, r2x, r2y, r2z)
    tbx, tby, tbz = _cross(-r2x, -r2y, -r2z, r4x, r4y, r4z)
    da2 = tax * tax + tay * tay + taz * taz
    db2 = tbx * tbx + tby * tby + tbz * tbz
    dot = tax * tbx + tay * tby + taz * tbz
    cp = dot * _rsqrt(jnp.maximum(da2 * db2, 1e-24))
    cp = jnp.clip(cp, -0.999999, 0.999999)
    sst = jnp.maximum(1.0 - cp * cp, 1e-12)
    cn2 = 1.0 - 2.0 * sst
    cn3 = cp * (cp * cp - 3.0 * sst)
    cn6 = 1.0 + sst * ((-32.0 * sst + 48.0) * sst - 18.0)
    cn = jnp.where(order == 2, cn2,
         jnp.where(order == 3, cn3,
         jnp.where(order == 6, cn6, 0.0)))
    return 0.5 * kf * (1.0 - cos_t * cn)


def _e_inversion(g, p):
    (xc, yc, zc), (xa, ya, za), (xb, yb, zb), (xd, yd, zd) = g
    kf, c0, c1, c2 = p
    rax = xa - xc; ray = ya - yc; raz = za - zc
    rbx = xb - xc; rby = yb - yc; rbz = zb - zc
    rdx = xd - xc; rdy = yd - yc; rdz = zd - zc
    nx, ny, nz = _cross(rax, ray, raz, rbx, rby, rbz)
    nn = nx * nx + ny * ny + nz * nz
    dd = rdx * rdx + rdy * rdy + rdz * rdz
    dot = nx * rdx + ny * rdy + nz * rdz
    sw = dot * _rsqrt(jnp.maximum(nn * dd, 1e-24))
    sw = jnp.clip(sw, -0.999999, 0.999999)
    z = jnp.maximum(1.0 - sw * sw, 1e-12)
    cw = z * _rsqrt(z)
    c2w = 2.0 * cw * cw - 1.0
    return kf * (c0 + c1 * cw + c2 * c2w)


def _e_vdw(g, p):
    (x0, y0, z0), (x1, y1, z1) = g
    rm, eps, thr = p
    dx = x0 - x1; dy = y0 - y1; dz = z0 - z1
    d2 = dx * dx + dy * dy + dz * dz
    r2c = jnp.maximum(d2, 0.01)
    t = rm * _rsqrt(r2c)
    x2 = t * t
    x6 = x2 * x2 * x2
    ev = eps * (x6 * x6 - 2.0 * x6)
    return jnp.where(r2c <= thr * thr, ev, 0.0)


def _pad_to(n, cw):
    q = 2 * NW * cw
    return ((n + q - 1) // q) * q


def _prep_idx(col, tot, n_atoms):
    pad = tot - col.shape[0]
    filler = jnp.arange(pad, dtype=jnp.int32) % n_atoms
    return jnp.concatenate([col.astype(jnp.int32), filler])


def _prep_par(a, tot):
    pad = tot - a.shape[0]
    return jnp.concatenate([a, jnp.zeros((pad,), a.dtype)])


_SECTIONS = (
    (2, 2, 0, CB, _e_bond),
    (3, 4, 1, CA, _e_angle),
    (4, 2, 1, CT, lambda g, p: _e_torsion(g, (p[0], p[2], p[1]))),
    (4, 4, 0, CI, _e_inversion),
    (2, 3, 0, CN, _e_vdw),
)


def _mk_kernel(counts, npad):
    nks = [_pad_to(n, s[3]) // (NW * s[3]) for n, s in zip(counts, _SECTIONS)]
    mesh = plsc.VectorSubcoreMesh(
        core_axis_name="c", subcore_axis_name="s",
        num_cores=NC, num_subcores=NS)

    def body(*refs):
        xs, ys, zs = refs[0:3]
        pos = 3
        sec_in = []
        for ni, nf, nint, _, _ in _SECTIONS:
            k = ni + nf + nint
            sec_in.append(refs[pos:pos + k])
            pos += k
        out = refs[pos]; pos += 1
        sec_scr = []
        for ni, nf, nint, _, _ in _SECTIONS:
            k = 2 * (4 * ni + nf + nint)
            sec_scr.append(refs[pos:pos + k])
            pos += k
        xsh, ysh, zsh = refs[pos:pos + 3]; pos += 3
        bounce = refs[pos]; pos += 1
        acc_v = refs[pos]; pos += 1
        sems = refs[pos:pos + 2]

        sid = lax.axis_index("s")
        wid = sid * NC + lax.axis_index("c")
        acc = jnp.zeros((L,), jnp.float32)

        npad = xsh.shape[0]
        stripe = npad // NS
        s0 = sid * stripe
        for h, v in ((xs, xsh), (ys, ysh), (zs, zsh)):
            pltpu.sync_copy(h.at[pl.ds(s0, stripe)], bounce)
            pltpu.sync_copy(bounce, v.at[pl.ds(s0, stripe)])
        plsc.subcore_barrier()

        for (ni, nf, nint, cw, efn), ins, scr, nk in zip(
                _SECTIONS, sec_in, sec_scr, nks):
            idx_h = ins[:ni]
            par_h = ins[ni:]
            half = 4 * ni + nf + nint
            sets = []
            for p in (0, 1):
                s = scr[p * half:(p + 1) * half]
                sets.append((s[:ni], s[ni:4 * ni], s[4 * ni:]))
            base_el = wid * (nk * cw)

            def fire_l(ci, p, idx_h=idx_h, par_h=par_h, sets=sets, cw=cw,
                       base_el=base_el):
                e0 = base_el + ci * cw
                idx_v, _, par_v = sets[p]
                for h, v in zip(idx_h + par_h, idx_v + par_v):
                    pltpu.async_copy(h.at[pl.ds(e0, cw)], v, sems[p])

            def wait_l(p, idx_h=idx_h, par_h=par_h, sets=sets, cw=cw,
                       base_el=base_el):
                idx_v, _, par_v = sets[p]
                for h, v in zip(idx_h + par_h, idx_v + par_v):
                    pltpu.make_async_copy(
                        h.at[pl.ds(base_el, cw)], v, sems[p]).wait()

            def compute(p, acc_c, sets=sets, cw=cw, ni=ni, efn=efn):
                _, g_v, par_v = sets[p]

                def slc(s, acc_r):
                    off = s * L
                    sl = lambda ref: ref[pl.ds(off, L)]
                    sites = tuple(
                        (sl(g_v[3 * j]), sl(g_v[3 * j + 1]),
                         sl(g_v[3 * j + 2]))
                        for j in range(ni))
                    pars = tuple(sl(v) for v in par_v)
                    return acc_r + efn(sites, pars)

                return lax.fori_loop(0, cw // L, slc, acc_c)

            fire_l(0, 0)
            fire_l(1, 1)
            nt = nk // 2

            def pair(t, acc_p, sets=sets, nk=nk):
                gds = []
                for p in (0, 1):
                    wait_l(p)
                    idx_v, g_v, _ = sets[p]
                    g = []
                    for j, iv in enumerate(idx_v):
                        for c, tbl in enumerate((xsh, ysh, zsh)):
                            g.append(pltpu.async_copy(
                                tbl.at[iv], g_v[3 * j + c], sems[p]))
                    gds.append(g)
                for p in (0, 1):
                    for d in gds[p]:
                        d.wait()
                    acc_p = compute(p, acc_p)
                    ci_next = jnp.minimum(2 * t + p + 2, nk - 1)
                    fire_l(ci_next, p)
                return acc_p

            acc = lax.fori_loop(0, nt, pair, acc)
            wait_l(0)
            wait_l(1)

        acc_v[...] = acc
        pltpu.sync_copy(acc_v, out.at[wid])

    scratch = []
    for ni, nf, nint, cw, _ in _SECTIONS:
        for _ in range(2):
            scratch += [pltpu.VMEM((cw,), jnp.int32) for _ in range(ni)]
            scratch += [pltpu.VMEM((cw,), jnp.float32) for _ in range(3 * ni)]
            scratch += [pltpu.VMEM((cw,), jnp.float32) for _ in range(nf)]
            scratch += [pltpu.VMEM((cw,), jnp.int32) for _ in range(nint)]
    scratch += [pltpu.VMEM_SHARED((npad,), jnp.float32) for _ in range(3)]
    scratch += [pltpu.VMEM((npad // NS,), jnp.float32)]
    scratch += [pltpu.VMEM((L,), jnp.float32)]
    scratch += [pltpu.SemaphoreType.DMA, pltpu.SemaphoreType.DMA]

    return pl.kernel(
        body,
        out_type=jax.ShapeDtypeStruct((NW, L), jnp.float32),
        mesh=mesh,
        scratch_types=scratch,
    )


def kernel(coords, bond_index, bond_rest_length, bond_force_constant,
           angle_index, angle_force_constant, angle_c0, angle_c1, angle_c2,
           angle_order, torsion_index, torsion_force_constant, torsion_order,
           torsion_cos_term, inversion_index, inversion_force_constant,
           inversion_c0, inversion_c1, inversion_c2, nonbond_index,
           vdw_minimum, vdw_well_depth, vdw_threshold):
    n_atoms = coords.shape[0]
    npad = ((n_atoms + 8 * NS - 1) // (8 * NS)) * (8 * NS)

    sec_arrays = (
        (bond_index, (bond_rest_length, bond_force_constant)),
        (angle_index, (angle_force_constant, angle_c0, angle_c1, angle_c2,
                       angle_order)),
        (torsion_index, (torsion_force_constant, torsion_cos_term,
                         torsion_order)),
        (inversion_index, (inversion_force_constant, inversion_c0,
                           inversion_c1, inversion_c2)),
        (nonbond_index, (vdw_minimum, vdw_well_depth, vdw_threshold)),
    )

    args = [_prep_par(coords[:, 0], npad),
            _prep_par(coords[:, 1], npad),
            _prep_par(coords[:, 2], npad)]
    counts = []
    for (idx2d, params), (ni, nf, nint, cw, _) in zip(sec_arrays, _SECTIONS):
        n = idx2d.shape[0]
        counts.append(n)
        tot = _pad_to(n, cw)
        for j in range(ni):
            args.append(_prep_idx(idx2d[:, j], tot, n_atoms))
        for par in params:
            args.append(_prep_par(par, tot))

    out = _mk_kernel(tuple(counts), npad)(*args)
    return jnp.sum(out)
